# baseline (speedup 1.0000x reference)
import functools

import jax
import jax.numpy as jnp
from jax import lax
from jax.experimental import pallas as pl
from jax.experimental.pallas import tpu as pltpu
from jax.experimental.pallas import tpu_sc as plsc

D_MODEL = 2048
D_FF_LIGHT = 512
D_FF_HEAVY = 8192

BLK_L = 1024
BLK_HA = 512
CH_FF = 2048
BLK_HB = 512
CH_DM = 1024

_INTERPRET = False


def _gelu_exact(h):
    return 0.5 * h * (1.0 + jax.lax.erf(h * 0.7071067811865476))


def _light_body(x_ref, wr_ref, w1_ref, w2_ref, out_ref, scores_ref):
    xf = x_ref[...]
    xb = xf.astype(jnp.bfloat16)
    wrow = wr_ref[...].astype(jnp.bfloat16).astype(jnp.float32)
    s = jnp.sum(xb.astype(jnp.float32) * wrow, axis=1, keepdims=True)
    scores_ref[...] = s
    h = jnp.dot(xb, w1_ref[...], preferred_element_type=jnp.float32)
    g = _gelu_exact(h).astype(jnp.bfloat16)
    out_ref[...] = jnp.dot(g, w2_ref[...], preferred_element_type=jnp.float32)


def _light_call(xf, w_router_t, w1b, w2b):
    n = xf.shape[0]
    grid = (n // BLK_L,)
    return pl.pallas_call(
        _light_body,
        grid=grid,
        in_specs=[
            pl.BlockSpec((BLK_L, D_MODEL), lambda i: (i, 0)),
            pl.BlockSpec((1, D_MODEL), lambda i: (0, 0)),
            pl.BlockSpec((D_MODEL, D_FF_LIGHT), lambda i: (0, 0)),
            pl.BlockSpec((D_FF_LIGHT, D_MODEL), lambda i: (0, 0)),
        ],
        out_specs=[
            pl.BlockSpec((BLK_L, D_MODEL), lambda i: (i, 0)),
            pl.BlockSpec((BLK_L, 1), lambda i: (i, 0)),
        ],
        out_shape=[
            jax.ShapeDtypeStruct((n, D_MODEL), jnp.float32),
            jax.ShapeDtypeStruct((n, 1), jnp.float32),
        ],
        interpret=_INTERPRET,
    )(xf, w_router_t, w1b, w2b)


def _topk_sc_call(scores, B, T, K):
    mesh = plsc.VectorSubcoreMesh(core_axis_name="c", subcore_axis_name="s")

    @functools.partial(
        pl.kernel,
        out_type=[
            jax.ShapeDtypeStruct((B, K), jnp.int32),
            jax.ShapeDtypeStruct((B, K), jnp.float32),
        ],
        mesh=mesh,
        scratch_types=[
            pltpu.VMEM((T,), jnp.float32),
            pltpu.VMEM((T,), jnp.int32),
            pltpu.VMEM((K + 16,), jnp.int32),
            pltpu.VMEM((K + 16,), jnp.float32),
        ],
        compiler_params=pltpu.CompilerParams(needs_layout_passes=False),
    )
    def topk_kernel(scores_ref, idx_ref, gates_ref, sc_v, key_v, idx_v, gate_v):
        c = lax.axis_index("c")
        s = lax.axis_index("s")

        @pl.when((c == 0) & (s < B))
        def _():
            row = s
            pltpu.sync_copy(scores_ref.at[row], sc_v)

            nvec = T // 16

            def mk(i, carry):
                v = sc_v[pl.ds(i * 16, 16)]
                b = jax.lax.bitcast_convert_type(v, jnp.int32)
                m = jax.lax.shift_right_arithmetic(b, 31)
                key_v[pl.ds(i * 16, 16)] = b ^ (m & jnp.int32(0x7FFFFFFF))
                return carry

            lax.fori_loop(0, nvec, mk, jnp.int32(0))

            onef = jnp.full((16,), 1.0, jnp.float32)
            zerof = jnp.full((16,), 0.0, jnp.float32)
            kf = jnp.float32(K)

            def count_gt(t):
                def cbody(i, cnt):
                    kv = key_v[pl.ds(i * 16, 16)]
                    return cnt + jnp.where(kv > t, onef, zerof)
                cntv = lax.fori_loop(0, nvec, cbody, zerof)
                return jnp.sum(cntv)

            def bis(_, lohi):
                lo, hi = lohi
                mid = (lo & hi) + ((lo ^ hi) >> 1)
                p = count_gt(mid)
                big = p >= kf
                return (jnp.where(big, mid + 1, lo), jnp.where(big, hi, mid))

            lo, _hi = lax.fori_loop(
                0, 32, bis,
                (jnp.int32(-2147483648), jnp.int32(2147483647)))
            vstar = lo
            budget = kf - count_gt(vstar)

            def sbody(i, carry):
                pos, ties = carry
                kv = key_v[pl.ds(i * 16, 16)]
                sv = sc_v[pl.ds(i * 16, 16)]
                sel_gt = kv > vstar
                sel_eq = kv == vstar
                eq_f = jnp.where(sel_eq, onef, zerof)
                pref = plsc.cumsum(eq_f)
                allowed = sel_eq & ((ties + pref) <= budget)
                sel = sel_gt | allowed
                gidx = (row * T + i * 16) + jax.lax.iota(jnp.int32, 16)
                plsc.store_compressed(idx_v.at[pl.ds(pos, 16)], gidx, mask=sel)
                gate = 1.0 / (1.0 + jnp.exp(-sv))
                plsc.store_compressed(gate_v.at[pl.ds(pos, 16)], gate, mask=sel)
                nsel = jnp.sum(jnp.where(sel, onef, zerof))
                neq = jnp.sum(eq_f)
                return (pos + nsel.astype(jnp.int32), ties + neq)

            lax.fori_loop(0, nvec, sbody, (jnp.int32(0), jnp.float32(0.0)))

            pltpu.sync_copy(idx_v.at[pl.ds(0, K)], idx_ref.at[row])
            pltpu.sync_copy(gate_v.at[pl.ds(0, K)], gates_ref.at[row])

    return topk_kernel(scores)


def _gather_sc_call(xf, light, idx_flat):
    n = idx_flat.shape[0]
    d = xf.shape[1]
    per_tile = n // 32
    nchunk = per_tile // 16
    mesh = plsc.VectorSubcoreMesh(core_axis_name="c", subcore_axis_name="s")

    @functools.partial(
        pl.kernel,
        out_type=[
            jax.ShapeDtypeStruct((n, d), jnp.float32),
            jax.ShapeDtypeStruct((n, d), jnp.float32),
        ],
        mesh=mesh,
        scratch_types=[
            pltpu.VMEM((per_tile,), jnp.int32),
            pltpu.VMEM((2, 16, d), jnp.float32),
            pltpu.SemaphoreType.DMA((2,)),
        ],
        compiler_params=pltpu.CompilerParams(needs_layout_passes=False),
    )
    def gather_kernel(x_ref, light_ref, idxf_ref, hin_ref, lsel_ref,
                      idx_v, buf_a, sem_a):
        c = lax.axis_index("c")
        s = lax.axis_index("s")
        wid = s * 2 + c
        base = wid * per_tile
        pltpu.sync_copy(idxf_ref.at[pl.ds(base, per_tile)], idx_v)

        def gather_pass(src_ref, dst_ref):
            def start(j, parity):
                return pltpu.async_copy(
                    src_ref.at[idx_v.at[pl.ds(j * 16, 16)]],
                    buf_a.at[parity],
                    sem_a.at[parity])

            start(0, 0)

            def chunk(j, carry):
                @pl.when(j + 1 < nchunk)
                def _():
                    start(j + 1, (j + 1) % 2)
                pltpu.make_async_copy(
                    src_ref.at[idx_v.at[pl.ds(j * 16, 16)]],
                    buf_a.at[j % 2],
                    sem_a.at[j % 2]).wait()
                pltpu.sync_copy(buf_a.at[j % 2],
                                dst_ref.at[pl.ds(base + j * 16, 16)])
                return carry

            lax.fori_loop(0, nchunk, chunk, jnp.int32(0))

        gather_pass(x_ref, hin_ref)
        gather_pass(light_ref, lsel_ref)

    return gather_kernel(xf, light, idx_flat)


def _heavy_a_body(x_ref, w1_ref, h_ref):
    xb = x_ref[...].astype(jnp.bfloat16)
    h = jnp.dot(xb, w1_ref[...], preferred_element_type=jnp.float32)
    h_ref[...] = _gelu_exact(h).astype(jnp.bfloat16)


def _heavy_a_call(hx, w1b):
    n = hx.shape[0]
    return pl.pallas_call(
        _heavy_a_body,
        grid=(D_FF_HEAVY // CH_FF, n // BLK_HA),
        in_specs=[
            pl.BlockSpec((BLK_HA, D_MODEL), lambda j, i: (i, 0)),
            pl.BlockSpec((D_MODEL, CH_FF), lambda j, i: (0, j)),
        ],
        out_specs=pl.BlockSpec((BLK_HA, CH_FF), lambda j, i: (i, j)),
        out_shape=jax.ShapeDtypeStruct((n, D_FF_HEAVY), jnp.bfloat16),
        interpret=_INTERPRET,
    )(hx, w1b)


def _make_heavy_b_body(ni, nsteps):
    def _heavy_b_body(idx_sref, h_ref, w2_ref, lsel_ref, gate_ref, light_ref,
                      out_ref, obuf_ref, sem):
        j = pl.program_id(0)
        i = pl.program_id(1)
        step = j * ni + i

        def drain():
            def wbody(r, carry):
                pltpu.make_async_copy(
                    obuf_ref.at[0], out_ref.at[0, pl.ds(0, CH_DM)],
                    sem).wait()
                return carry
            lax.fori_loop(0, BLK_HB, wbody, jnp.int32(0))

        @pl.when(step > 0)
        def _():
            drain()

        o = jnp.dot(h_ref[...], w2_ref[...],
                    preferred_element_type=jnp.float32)
        o = o * gate_ref[...] + lsel_ref[...]
        obuf_ref[...] = o

        def rbody(r, carry):
            gid = idx_sref[i * BLK_HB + r]
            pltpu.make_async_copy(
                obuf_ref.at[r],
                out_ref.at[gid, pl.ds(j * CH_DM, CH_DM)], sem).start()
            return carry

        lax.fori_loop(0, BLK_HB, rbody, jnp.int32(0))

        @pl.when(step == nsteps - 1)
        def _():
            drain()

    return _heavy_b_body


def _heavy_b_call(idx_flat, h, w2b, gates, lsel, light):
    n = h.shape[0]
    nj = D_MODEL // CH_DM
    ni = n // BLK_HB
    grid_spec = pltpu.PrefetchScalarGridSpec(
        num_scalar_prefetch=1,
        grid=(nj, ni),
        in_specs=[
            pl.BlockSpec((BLK_HB, D_FF_HEAVY), lambda j, i, idx: (i, 0)),
            pl.BlockSpec((D_FF_HEAVY, CH_DM), lambda j, i, idx: (0, j)),
            pl.BlockSpec((BLK_HB, CH_DM), lambda j, i, idx: (i, j)),
            pl.BlockSpec((BLK_HB, 1), lambda j, i, idx: (i, 0)),
            pl.BlockSpec(memory_space=pl.ANY),
        ],
        out_specs=pl.BlockSpec(memory_space=pl.ANY),
        scratch_shapes=[
            pltpu.VMEM((BLK_HB, CH_DM), jnp.float32),
            pltpu.SemaphoreType.DMA,
        ],
    )
    return pl.pallas_call(
        _make_heavy_b_body(ni, nj * ni),
        grid_spec=grid_spec,
        out_shape=jax.ShapeDtypeStruct(light.shape, jnp.float32),
        input_output_aliases={5: 0},
        compiler_params=pltpu.CompilerParams(
            vmem_limit_bytes=66_000_000),
        interpret=_INTERPRET,
    )(idx_flat, h, w2b, lsel, gates, light)


def kernel(x, w_router, w_l1, w_l2, w_h1, w_h2):
    B, T, d = x.shape
    k = T // 4
    xf = x.reshape(B * T, d)
    w1b = w_l1.astype(jnp.bfloat16)
    w2b = w_l2.astype(jnp.bfloat16)
    wh1b = w_h1.astype(jnp.bfloat16)
    wh2b = w_h2.astype(jnp.bfloat16)

    light, scores2 = _light_call(xf, w_router.T, w1b, w2b)
    scores = scores2.reshape(B, T)

    idx, gates = _topk_sc_call(scores, B, T, k)
    idx_flat = idx.reshape(B * k)

    heavy_in, lsel = _gather_sc_call(xf, light, idx_flat)
    h = _heavy_a_call(heavy_in, wh1b)
    out = _heavy_b_call(idx_flat, h, wh2b, gates.reshape(B * k, 1), lsel,
                        light)
    return (out.reshape(B, T, d), scores)


# speedup vs baseline: 1.1196x; 1.1196x over previous
import functools

import jax
import jax.numpy as jnp
from jax import lax
from jax.experimental import pallas as pl
from jax.experimental.pallas import tpu as pltpu
from jax.experimental.pallas import tpu_sc as plsc

D_MODEL = 2048
D_FF_LIGHT = 512
D_FF_HEAVY = 8192

BLK_L = 1024
BLK_HA = 512
CH_FF = 2048
BLK_HB = 512
CH_DM = 1024

_INTERPRET = False


def _gelu_exact(h):
    return 0.5 * h * (1.0 + jax.lax.erf(h * 0.7071067811865476))


def _light_body(x_ref, wr_ref, w1_ref, w2_ref, out_ref, scores_ref):
    xf = x_ref[...]
    xb = xf.astype(jnp.bfloat16)
    wrow = wr_ref[...].astype(jnp.bfloat16).astype(jnp.float32)
    s = jnp.sum(xb.astype(jnp.float32) * wrow, axis=1, keepdims=True)
    scores_ref[...] = s
    h = jnp.dot(xb, w1_ref[...], preferred_element_type=jnp.float32)
    g = _gelu_exact(h).astype(jnp.bfloat16)
    out_ref[...] = jnp.dot(g, w2_ref[...], preferred_element_type=jnp.float32)


def _light_call(xf, w_router_t, w1b, w2b):
    n = xf.shape[0]
    grid = (n // BLK_L,)
    return pl.pallas_call(
        _light_body,
        grid=grid,
        in_specs=[
            pl.BlockSpec((BLK_L, D_MODEL), lambda i: (i, 0)),
            pl.BlockSpec((1, D_MODEL), lambda i: (0, 0)),
            pl.BlockSpec((D_MODEL, D_FF_LIGHT), lambda i: (0, 0)),
            pl.BlockSpec((D_FF_LIGHT, D_MODEL), lambda i: (0, 0)),
        ],
        out_specs=[
            pl.BlockSpec((BLK_L, D_MODEL), lambda i: (i, 0)),
            pl.BlockSpec((BLK_L, 1), lambda i: (i, 0)),
        ],
        out_shape=[
            jax.ShapeDtypeStruct((n, D_MODEL), jnp.float32),
            jax.ShapeDtypeStruct((n, 1), jnp.float32),
        ],
        interpret=_INTERPRET,
    )(xf, w_router_t, w1b, w2b)


def _topk_sc_call(scores, B, T, K):
    mesh = plsc.VectorSubcoreMesh(core_axis_name="c", subcore_axis_name="s")

    @functools.partial(
        pl.kernel,
        out_type=[
            jax.ShapeDtypeStruct((B, K), jnp.int32),
            jax.ShapeDtypeStruct((B, K), jnp.float32),
        ],
        mesh=mesh,
        scratch_types=[
            pltpu.VMEM((T,), jnp.float32),
            pltpu.VMEM((T,), jnp.int32),
            pltpu.VMEM((K + 16,), jnp.int32),
            pltpu.VMEM((K + 16,), jnp.float32),
        ],
        compiler_params=pltpu.CompilerParams(needs_layout_passes=False),
    )
    def topk_kernel(scores_ref, idx_ref, gates_ref, sc_v, key_v, idx_v, gate_v):
        c = lax.axis_index("c")
        s = lax.axis_index("s")

        @pl.when((c == 0) & (s < B))
        def _():
            row = s
            pltpu.sync_copy(scores_ref.at[row], sc_v)

            nvec = T // 16

            def mk(i, carry):
                v = sc_v[pl.ds(i * 16, 16)]
                b = jax.lax.bitcast_convert_type(v, jnp.int32)
                m = jax.lax.shift_right_arithmetic(b, 31)
                key_v[pl.ds(i * 16, 16)] = b ^ (m & jnp.int32(0x7FFFFFFF))
                return carry

            lax.fori_loop(0, nvec, mk, jnp.int32(0))

            onef = jnp.full((16,), 1.0, jnp.float32)
            zerof = jnp.full((16,), 0.0, jnp.float32)
            kf = jnp.float32(K)

            def count_gt(t):
                def cbody(i, cnt):
                    kv = key_v[pl.ds(i * 16, 16)]
                    return cnt + jnp.where(kv > t, onef, zerof)
                cntv = lax.fori_loop(0, nvec, cbody, zerof)
                return jnp.sum(cntv)

            def bis(_, lohi):
                lo, hi = lohi
                mid = (lo & hi) + ((lo ^ hi) >> 1)
                p = count_gt(mid)
                big = p >= kf
                return (jnp.where(big, mid + 1, lo), jnp.where(big, hi, mid))

            lo, _hi = lax.fori_loop(
                0, 32, bis,
                (jnp.int32(-2147483648), jnp.int32(2147483647)))
            vstar = lo
            budget = kf - count_gt(vstar)

            def sbody(i, carry):
                pos, ties = carry
                kv = key_v[pl.ds(i * 16, 16)]
                sv = sc_v[pl.ds(i * 16, 16)]
                sel_gt = kv > vstar
                sel_eq = kv == vstar
                eq_f = jnp.where(sel_eq, onef, zerof)
                pref = plsc.cumsum(eq_f)
                allowed = sel_eq & ((ties + pref) <= budget)
                sel = sel_gt | allowed
                gidx = (row * T + i * 16) + jax.lax.iota(jnp.int32, 16)
                plsc.store_compressed(idx_v.at[pl.ds(pos, 16)], gidx, mask=sel)
                gate = 1.0 / (1.0 + jnp.exp(-sv))
                plsc.store_compressed(gate_v.at[pl.ds(pos, 16)], gate, mask=sel)
                nsel = jnp.sum(jnp.where(sel, onef, zerof))
                neq = jnp.sum(eq_f)
                return (pos + nsel.astype(jnp.int32), ties + neq)

            lax.fori_loop(0, nvec, sbody, (jnp.int32(0), jnp.float32(0.0)))

            pltpu.sync_copy(idx_v.at[pl.ds(0, K)], idx_ref.at[row])
            pltpu.sync_copy(gate_v.at[pl.ds(0, K)], gates_ref.at[row])

    return topk_kernel(scores)


def _gather_sc_call(xf, light, idx_flat):
    n = idx_flat.shape[0]
    d = xf.shape[1]
    per_tile = n // 32
    nchunk = per_tile // 16
    mesh = plsc.VectorSubcoreMesh(core_axis_name="c", subcore_axis_name="s")

    @functools.partial(
        pl.kernel,
        out_type=[
            jax.ShapeDtypeStruct((n, d), jnp.float32),
            jax.ShapeDtypeStruct((n, d), jnp.float32),
        ],
        mesh=mesh,
        scratch_types=[
            pltpu.VMEM((per_tile,), jnp.int32),
            pltpu.VMEM((2, 16, d), jnp.float32),
            pltpu.SemaphoreType.DMA((2,)),
        ],
        compiler_params=pltpu.CompilerParams(needs_layout_passes=False),
    )
    def gather_kernel(x_ref, light_ref, idxf_ref, hin_ref, lsel_ref,
                      idx_v, buf_a, sem_a):
        c = lax.axis_index("c")
        s = lax.axis_index("s")
        wid = s * 2 + c
        base = wid * per_tile
        pltpu.sync_copy(idxf_ref.at[pl.ds(base, per_tile)], idx_v)

        def gather_pass(src_ref, dst_ref):
            def start(j, parity):
                return pltpu.async_copy(
                    src_ref.at[idx_v.at[pl.ds(j * 16, 16)]],
                    buf_a.at[parity],
                    sem_a.at[parity])

            start(0, 0)

            def chunk(j, carry):
                @pl.when(j + 1 < nchunk)
                def _():
                    start(j + 1, (j + 1) % 2)
                pltpu.make_async_copy(
                    src_ref.at[idx_v.at[pl.ds(j * 16, 16)]],
                    buf_a.at[j % 2],
                    sem_a.at[j % 2]).wait()
                pltpu.sync_copy(buf_a.at[j % 2],
                                dst_ref.at[pl.ds(base + j * 16, 16)])
                return carry

            lax.fori_loop(0, nchunk, chunk, jnp.int32(0))

        gather_pass(x_ref, hin_ref)
        gather_pass(light_ref, lsel_ref)

    return gather_kernel(xf, light, idx_flat)


def _heavy_a_body(x_ref, w1_ref, h_ref):
    xb = x_ref[...].astype(jnp.bfloat16)
    h = jnp.dot(xb, w1_ref[...], preferred_element_type=jnp.float32)
    h_ref[...] = _gelu_exact(h).astype(jnp.bfloat16)


def _heavy_a_call(hx, w1b):
    n = hx.shape[0]
    return pl.pallas_call(
        _heavy_a_body,
        grid=(D_FF_HEAVY // CH_FF, n // BLK_HA),
        in_specs=[
            pl.BlockSpec((BLK_HA, D_MODEL), lambda j, i: (i, 0)),
            pl.BlockSpec((D_MODEL, CH_FF), lambda j, i: (0, j)),
        ],
        out_specs=pl.BlockSpec((BLK_HA, CH_FF), lambda j, i: (i, j)),
        out_shape=jax.ShapeDtypeStruct((n, D_FF_HEAVY), jnp.bfloat16),
        interpret=_INTERPRET,
    )(hx, w1b)


BLK_B = 256


def _make_heavy_b_body(nblk):
    def _heavy_b_body(idx_sref, h_ref, w2_ref, lsel_ref, gate_ref, light_ref,
                      out_ref, obuf_ref, sem):
        i = pl.program_id(0)
        slot = i % 2

        @pl.when(i > 1)
        def _():
            def wbody(r, carry):
                pltpu.make_async_copy(
                    obuf_ref.at[0, 0], out_ref.at[0], sem).wait()
                return carry
            lax.fori_loop(0, BLK_B, wbody, jnp.int32(0))

        o = jnp.dot(h_ref[...], w2_ref[...],
                    preferred_element_type=jnp.float32)
        o = o * gate_ref[...] + lsel_ref[...]
        obuf_ref[slot] = o

        def rbody(r, carry):
            gid = idx_sref[i * BLK_B + r]
            pltpu.make_async_copy(
                obuf_ref.at[slot, r], out_ref.at[gid], sem).start()
            return carry

        lax.fori_loop(0, BLK_B, rbody, jnp.int32(0))

        @pl.when(i == nblk - 1)
        def _():
            def wbody(r, carry):
                pltpu.make_async_copy(
                    obuf_ref.at[0, 0], out_ref.at[0], sem).wait()
                return carry
            nleft = 2 * BLK_B if nblk > 1 else BLK_B
            lax.fori_loop(0, nleft, wbody, jnp.int32(0))

    return _heavy_b_body


def _heavy_b_call(idx_flat, h, w2b, gates, lsel, light):
    n = h.shape[0]
    grid_spec = pltpu.PrefetchScalarGridSpec(
        num_scalar_prefetch=1,
        grid=(n // BLK_B,),
        in_specs=[
            pl.BlockSpec((BLK_B, D_FF_HEAVY), lambda i, idx: (i, 0)),
            pl.BlockSpec((D_FF_HEAVY, D_MODEL), lambda i, idx: (0, 0)),
            pl.BlockSpec((BLK_B, D_MODEL), lambda i, idx: (i, 0)),
            pl.BlockSpec((BLK_B, 1), lambda i, idx: (i, 0)),
            pl.BlockSpec(memory_space=pl.ANY),
        ],
        out_specs=pl.BlockSpec(memory_space=pl.ANY),
        scratch_shapes=[
            pltpu.VMEM((2, BLK_B, D_MODEL), jnp.float32),
            pltpu.SemaphoreType.DMA,
        ],
    )
    return pl.pallas_call(
        _make_heavy_b_body(n // BLK_B),
        grid_spec=grid_spec,
        out_shape=jax.ShapeDtypeStruct(light.shape, jnp.float32),
        input_output_aliases={5: 0},
        interpret=_INTERPRET,
    )(idx_flat, h, w2b, lsel, gates, light)


def kernel(x, w_router, w_l1, w_l2, w_h1, w_h2):
    B, T, d = x.shape
    k = T // 4
    xf = x.reshape(B * T, d)
    w1b = w_l1.astype(jnp.bfloat16)
    w2b = w_l2.astype(jnp.bfloat16)
    wh1b = w_h1.astype(jnp.bfloat16)
    wh2b = w_h2.astype(jnp.bfloat16)

    light, scores2 = _light_call(xf, w_router.T, w1b, w2b)
    scores = scores2.reshape(B, T)

    idx, gates = _topk_sc_call(scores, B, T, k)
    idx_flat = idx.reshape(B * k)

    heavy_in, lsel = _gather_sc_call(xf, light, idx_flat)
    h = _heavy_a_call(heavy_in, wh1b)
    out = _heavy_b_call(idx_flat, h, wh2b, gates.reshape(B * k, 1), lsel,
                        light)
    return (out.reshape(B, T, d), scores)


# speedup vs baseline: 1.1519x; 1.0288x over previous
import functools

import jax
import jax.numpy as jnp
from jax import lax
from jax.experimental import pallas as pl
from jax.experimental.pallas import tpu as pltpu
from jax.experimental.pallas import tpu_sc as plsc

D_MODEL = 2048
D_FF_LIGHT = 512
D_FF_HEAVY = 8192

BLK_L = 1024
BLK_HA = 512
CH_FF = 2048
BLK_HB = 512
CH_DM = 1024

_INTERPRET = False


def _gelu_exact(h):
    return 0.5 * h * (1.0 + jax.lax.erf(h * 0.7071067811865476))


def _light_body(x_ref, wr_ref, w1_ref, w2_ref, out_ref, scores_ref):
    xf = x_ref[...]
    xb = xf.astype(jnp.bfloat16)
    wrow = wr_ref[...].astype(jnp.bfloat16).astype(jnp.float32)
    s = jnp.sum(xb.astype(jnp.float32) * wrow, axis=1, keepdims=True)
    scores_ref[...] = s
    h = jnp.dot(xb, w1_ref[...], preferred_element_type=jnp.float32)
    g = _gelu_exact(h).astype(jnp.bfloat16)
    out_ref[...] = jnp.dot(g, w2_ref[...], preferred_element_type=jnp.float32)


def _light_call(xf, w_router_t, w1b, w2b):
    n = xf.shape[0]
    grid = (n // BLK_L,)
    return pl.pallas_call(
        _light_body,
        grid=grid,
        in_specs=[
            pl.BlockSpec((BLK_L, D_MODEL), lambda i: (i, 0)),
            pl.BlockSpec((1, D_MODEL), lambda i: (0, 0)),
            pl.BlockSpec((D_MODEL, D_FF_LIGHT), lambda i: (0, 0)),
            pl.BlockSpec((D_FF_LIGHT, D_MODEL), lambda i: (0, 0)),
        ],
        out_specs=[
            pl.BlockSpec((BLK_L, D_MODEL), lambda i: (i, 0)),
            pl.BlockSpec((BLK_L, 1), lambda i: (i, 0)),
        ],
        out_shape=[
            jax.ShapeDtypeStruct((n, D_MODEL), jnp.float32),
            jax.ShapeDtypeStruct((n, 1), jnp.float32),
        ],
        interpret=_INTERPRET,
    )(xf, w_router_t, w1b, w2b)


def _topk_sc_call(scores, B, T, K):
    mesh = plsc.VectorSubcoreMesh(core_axis_name="c", subcore_axis_name="s")

    @functools.partial(
        pl.kernel,
        out_type=[
            jax.ShapeDtypeStruct((B, K), jnp.int32),
            jax.ShapeDtypeStruct((B, K), jnp.float32),
        ],
        mesh=mesh,
        scratch_types=[
            pltpu.VMEM((T,), jnp.float32),
            pltpu.VMEM((T,), jnp.int32),
            pltpu.VMEM((K + 16,), jnp.int32),
            pltpu.VMEM((K + 16,), jnp.float32),
        ],
        compiler_params=pltpu.CompilerParams(needs_layout_passes=False),
    )
    def topk_kernel(scores_ref, idx_ref, gates_ref, sc_v, key_v, idx_v, gate_v):
        c = lax.axis_index("c")
        s = lax.axis_index("s")

        @pl.when((c == 0) & (s < B))
        def _():
            row = s
            pltpu.sync_copy(scores_ref.at[row], sc_v)

            nvec = T // 16

            def mk(i, carry):
                v = sc_v[pl.ds(i * 16, 16)]
                b = jax.lax.bitcast_convert_type(v, jnp.int32)
                m = jax.lax.shift_right_arithmetic(b, 31)
                key_v[pl.ds(i * 16, 16)] = b ^ (m & jnp.int32(0x7FFFFFFF))
                return carry

            lax.fori_loop(0, nvec, mk, jnp.int32(0))

            onef = jnp.full((16,), 1.0, jnp.float32)
            zerof = jnp.full((16,), 0.0, jnp.float32)
            kf = jnp.float32(K)

            def count_gt(t):
                def cbody(i, cnt):
                    for u in range(8):
                        kv = key_v[pl.ds(i * 128 + u * 16, 16)]
                        cnt = cnt + jnp.where(kv > t, onef, zerof)
                    return cnt
                cntv = lax.fori_loop(0, nvec // 8, cbody, zerof)
                return jnp.sum(cntv)

            def bis(_, lohi):
                lo, hi = lohi
                mid = (lo & hi) + ((lo ^ hi) >> 1)
                p = count_gt(mid)
                big = p >= kf
                return (jnp.where(big, mid + 1, lo), jnp.where(big, hi, mid))

            lo, _hi = lax.fori_loop(
                0, 32, bis,
                (jnp.int32(-2147483648), jnp.int32(2147483647)))
            vstar = lo
            budget = kf - count_gt(vstar)

            def sbody(i, carry):
                pos, ties = carry
                kv = key_v[pl.ds(i * 16, 16)]
                sv = sc_v[pl.ds(i * 16, 16)]
                sel_gt = kv > vstar
                sel_eq = kv == vstar
                eq_f = jnp.where(sel_eq, onef, zerof)
                pref = plsc.cumsum(eq_f)
                allowed = sel_eq & ((ties + pref) <= budget)
                sel = sel_gt | allowed
                gidx = (row * T + i * 16) + jax.lax.iota(jnp.int32, 16)
                plsc.store_compressed(idx_v.at[pl.ds(pos, 16)], gidx, mask=sel)
                gate = 1.0 / (1.0 + jnp.exp(-sv))
                plsc.store_compressed(gate_v.at[pl.ds(pos, 16)], gate, mask=sel)
                nsel = jnp.sum(jnp.where(sel, onef, zerof))
                neq = jnp.sum(eq_f)
                return (pos + nsel.astype(jnp.int32), ties + neq)

            lax.fori_loop(0, nvec, sbody, (jnp.int32(0), jnp.float32(0.0)))

            pltpu.sync_copy(idx_v.at[pl.ds(0, K)], idx_ref.at[row])
            pltpu.sync_copy(gate_v.at[pl.ds(0, K)], gates_ref.at[row])

    return topk_kernel(scores)


def _gather_sc_call(xf, light, idx_flat):
    n = idx_flat.shape[0]
    d = xf.shape[1]
    per_tile = n // 32
    nchunk = per_tile // 16
    mesh = plsc.VectorSubcoreMesh(core_axis_name="c", subcore_axis_name="s")

    @functools.partial(
        pl.kernel,
        out_type=[
            jax.ShapeDtypeStruct((n, d), jnp.float32),
            jax.ShapeDtypeStruct((n, d), jnp.float32),
        ],
        mesh=mesh,
        scratch_types=[
            pltpu.VMEM((per_tile,), jnp.int32),
            pltpu.VMEM((2, 16, d), jnp.float32),
            pltpu.SemaphoreType.DMA((2,)),
            pltpu.SemaphoreType.DMA((2,)),
        ],
        compiler_params=pltpu.CompilerParams(needs_layout_passes=False),
    )
    def gather_kernel(x_ref, light_ref, idxf_ref, hin_ref, lsel_ref,
                      idx_v, buf_a, sem_a, sem_w):
        c = lax.axis_index("c")
        s = lax.axis_index("s")
        wid = s * 2 + c
        base = wid * per_tile
        pltpu.sync_copy(idxf_ref.at[pl.ds(base, per_tile)], idx_v)

        def gather_pass(src_ref, dst_ref):
            def gdesc(j):
                return pltpu.make_async_copy(
                    src_ref.at[idx_v.at[pl.ds(j * 16, 16)]],
                    buf_a.at[j % 2],
                    sem_a.at[j % 2])

            def wdesc(j):
                return pltpu.make_async_copy(
                    buf_a.at[j % 2],
                    dst_ref.at[pl.ds(base + j * 16, 16)],
                    sem_w.at[j % 2])

            gdesc(0).start()

            def chunk(j, carry):
                gdesc(j).wait()
                wdesc(j).start()

                @pl.when(j + 1 < nchunk)
                def _():
                    @pl.when(j >= 1)
                    def _():
                        wdesc(j - 1).wait()
                    gdesc(j + 1).start()
                return carry

            lax.fori_loop(0, nchunk, chunk, jnp.int32(0))
            wdesc(nchunk - 2).wait()
            wdesc(nchunk - 1).wait()

        gather_pass(x_ref, hin_ref)
        gather_pass(light_ref, lsel_ref)

    return gather_kernel(xf, light, idx_flat)


def _heavy_a_body(x_ref, w1_ref, h_ref):
    xb = x_ref[...].astype(jnp.bfloat16)
    h = jnp.dot(xb, w1_ref[...], preferred_element_type=jnp.float32)
    h_ref[...] = _gelu_exact(h).astype(jnp.bfloat16)


def _heavy_a_call(hx, w1b):
    n = hx.shape[0]
    return pl.pallas_call(
        _heavy_a_body,
        grid=(D_FF_HEAVY // CH_FF, n // BLK_HA),
        in_specs=[
            pl.BlockSpec((BLK_HA, D_MODEL), lambda j, i: (i, 0)),
            pl.BlockSpec((D_MODEL, CH_FF), lambda j, i: (0, j)),
        ],
        out_specs=pl.BlockSpec((BLK_HA, CH_FF), lambda j, i: (i, j)),
        out_shape=jax.ShapeDtypeStruct((n, D_FF_HEAVY), jnp.bfloat16),
        interpret=_INTERPRET,
    )(hx, w1b)


BLK_B = 256


def _make_heavy_b_body(nblk):
    def _heavy_b_body(idx_sref, h_ref, w2_ref, lsel_ref, gate_ref, light_ref,
                      out_ref, obuf_ref, sem):
        i = pl.program_id(0)
        slot = i % 2

        @pl.when(i > 1)
        def _():
            def wbody(r, carry):
                pltpu.make_async_copy(
                    obuf_ref.at[0, 0], out_ref.at[0], sem).wait()
                return carry
            lax.fori_loop(0, BLK_B, wbody, jnp.int32(0))

        o = jnp.dot(h_ref[...], w2_ref[...],
                    preferred_element_type=jnp.float32)
        o = o * gate_ref[...] + lsel_ref[...]
        obuf_ref[slot] = o

        def rbody(r, carry):
            gid = idx_sref[i * BLK_B + r]
            pltpu.make_async_copy(
                obuf_ref.at[slot, r], out_ref.at[gid], sem).start()
            return carry

        lax.fori_loop(0, BLK_B, rbody, jnp.int32(0))

        @pl.when(i == nblk - 1)
        def _():
            def wbody(r, carry):
                pltpu.make_async_copy(
                    obuf_ref.at[0, 0], out_ref.at[0], sem).wait()
                return carry
            nleft = 2 * BLK_B if nblk > 1 else BLK_B
            lax.fori_loop(0, nleft, wbody, jnp.int32(0))

    return _heavy_b_body


def _heavy_b_call(idx_flat, h, w2b, gates, lsel, light):
    n = h.shape[0]
    grid_spec = pltpu.PrefetchScalarGridSpec(
        num_scalar_prefetch=1,
        grid=(n // BLK_B,),
        in_specs=[
            pl.BlockSpec((BLK_B, D_FF_HEAVY), lambda i, idx: (i, 0)),
            pl.BlockSpec((D_FF_HEAVY, D_MODEL), lambda i, idx: (0, 0)),
            pl.BlockSpec((BLK_B, D_MODEL), lambda i, idx: (i, 0)),
            pl.BlockSpec((BLK_B, 1), lambda i, idx: (i, 0)),
            pl.BlockSpec(memory_space=pl.ANY),
        ],
        out_specs=pl.BlockSpec(memory_space=pl.ANY),
        scratch_shapes=[
            pltpu.VMEM((2, BLK_B, D_MODEL), jnp.float32),
            pltpu.SemaphoreType.DMA,
        ],
    )
    return pl.pallas_call(
        _make_heavy_b_body(n // BLK_B),
        grid_spec=grid_spec,
        out_shape=jax.ShapeDtypeStruct(light.shape, jnp.float32),
        input_output_aliases={5: 0},
        interpret=_INTERPRET,
    )(idx_flat, h, w2b, lsel, gates, light)


def kernel(x, w_router, w_l1, w_l2, w_h1, w_h2):
    B, T, d = x.shape
    k = T // 4
    xf = x.reshape(B * T, d)
    w1b = w_l1.astype(jnp.bfloat16)
    w2b = w_l2.astype(jnp.bfloat16)
    wh1b = w_h1.astype(jnp.bfloat16)
    wh2b = w_h2.astype(jnp.bfloat16)

    light, scores2 = _light_call(xf, w_router.T, w1b, w2b)
    scores = scores2.reshape(B, T)

    idx, gates = _topk_sc_call(scores, B, T, k)
    idx_flat = idx.reshape(B * k)

    heavy_in, lsel = _gather_sc_call(xf, light, idx_flat)
    h = _heavy_a_call(heavy_in, wh1b)
    out = _heavy_b_call(idx_flat, h, wh2b, gates.reshape(B * k, 1), lsel,
                        light)
    return (out.reshape(B, T, d), scores)
